# Initial kernel scaffold; baseline (speedup 1.0000x reference)
import jax
import jax.numpy as jnp
from jax.experimental import pallas as pl


def kernel(Q_in, K_in, V_in, t, W_v):
    raise NotImplementedError("write your pallas kernel here")


# speedup vs baseline: 30.6916x; 30.6916x over previous
import functools

import numpy as np
import jax
import jax.numpy as jnp
from jax.experimental import pallas as pl
from jax.experimental.pallas import tpu as pltpu

T = 288
NF = T // 2 + 1
NF2 = 2 * NF
FPH = 16
H = 8
TOPK = 5
F = 128
NB = 23
NPB = 207


def _dft_mats():
    f = np.arange(NF)
    tt = np.arange(T)
    ang = 2.0 * np.pi * np.outer(tt, f) / T
    cr = np.cos(ang)
    ci = -np.sin(ang)
    crci = np.concatenate([cr, ci], axis=1).astype(np.float32)
    wf = np.full(NF, 2.0)
    wf[0] = 1.0
    wf[-1] = 1.0
    drt = wf[:, None] * np.cos(ang.T) / T
    dit = -wf[:, None] * np.sin(ang.T) / T
    drdit = np.concatenate([drt, dit], axis=0).astype(np.float32)

    def split(m):
        import ml_dtypes
        hi = m.astype(np.float32).astype(ml_dtypes.bfloat16)
        mid = (m.astype(np.float32) - hi.astype(np.float32)).astype(ml_dtypes.bfloat16)
        return hi, mid

    ch, cm = split(crci)
    dh, dm = split(drdit)
    d2h, d2m = split(np.ascontiguousarray(drdit.T))
    return ch, cm, dh, dm, d2h, d2m


def _dot16(a, b):
    return jnp.dot(a, b, preferred_element_type=jnp.float32)


def _bsplit(x):
    xh = x.astype(jnp.bfloat16)
    xm = (x - xh.astype(jnp.float32)).astype(jnp.bfloat16)
    return xh, xm


def _dot4(x, bh, bm):
    xh, xm = _bsplit(x)
    return ((_dot16(xm, bm) + _dot16(xm, bh)) + _dot16(xh, bm)) + _dot16(xh, bh)


def _dot3(x, bh, bm):
    xh, xm = _bsplit(x)
    return (_dot16(xm, bh) + _dot16(xh, bm)) + _dot16(xh, bh)


def _t_nodes(x):
    return jnp.transpose(x, (0, 2, 1)).reshape(x.shape[0] * x.shape[2], T)


WAVES = ((0, 23),)


def _kern(q_ref, k_ref, v_ref, t_ref, wv_ref, ch_ref, cm_ref, dh_ref, dm_ref,
          d2h_ref, d2m_ref, out_ref, delay_ref, w_ref):
    w16 = wv_ref[0:F, :].astype(jnp.bfloat16)
    t16 = t_ref[0, 0, :].astype(jnp.bfloat16).astype(jnp.float32)
    w_row = wv_ref[F:F + 1, :].astype(jnp.bfloat16).astype(jnp.float32)
    tterm = t16[:, None] * w_row
    ch = ch_ref[...]
    cm = cm_ref[...]
    dh = dh_ref[...]
    dm = dm_ref[...]

    def stage_corr(a, b):
        nb = b - a
        r = nb * FPH

        def proj(x_ref):
            x16 = x_ref[a:b].reshape(nb * T, F).astype(jnp.bfloat16)
            p = _dot16(x16, w16).reshape(nb, T, FPH) + tterm[None]
            return _t_nodes(p)

        qT = proj(q_ref)
        kT = proj(k_ref)
        qf = _dot4(qT, ch, cm)
        kf = _dot4(kT, ch, cm)
        vT = _t_nodes(v_ref[a:b])
        vf = _dot3(vT, ch, cm)
        qfr, qfi = qf[:, :NF], qf[:, NF:]
        kfr, kfi = kf[:, :NF], kf[:, NF:]
        pr = qfr * kfr + qfi * kfi
        pi = qfi * kfr - qfr * kfi
        pf = jnp.concatenate([pr, pi], axis=1)
        corr = _dot4(pf, dh, dm)
        return corr, vf

    def stage_topk(corr, a, b):
        nb = b - a
        r = nb * FPH
        lane = jax.lax.broadcasted_iota(jnp.int32, (r, T), 1)
        c = corr
        ws, ds = [], []
        sx = jnp.zeros((r, T), jnp.float32)
        m0 = None
        for _ in range(TOPK):
            m = jnp.max(c, axis=1, keepdims=True)
            ix = jnp.argmax(c, axis=1)[:, None].astype(jnp.int32)
            ws.append(m)
            ds.append(ix)
            if m0 is None:
                m0 = m
            ei = jnp.exp(m - m0)
            msk = lane == ix
            sx = sx + jnp.where(msk, ei, 0.0)
            c = jnp.where(msk, -jnp.inf, c)
        w5 = jnp.concatenate(ws, axis=1)
        d5 = jnp.concatenate(ds, axis=1)
        e = jnp.exp(w5 - m0)
        z = jnp.sum(e, axis=1, keepdims=True)
        sm = e / z
        delay_ref[a * FPH:b * FPH] = d5
        w_ref[a * FPH:b * FPH] = sm
        return sx / z

    def stage_agg(s, vf, a, b):
        nb = b - a
        r = nb * FPH
        rw = jax.lax.broadcasted_iota(jnp.int32, (nb, r), 0)
        cl = jax.lax.broadcasted_iota(jnp.int32, (nb, r), 1)
        g16 = jnp.where((cl // FPH == rw) & (cl % FPH < H),
                        1.0 / H, 0.0).astype(jnp.bfloat16)
        alpha = _dot16(g16, s.astype(jnp.bfloat16))

        af = _dot3(alpha, ch, cm)
        afr_b = jnp.broadcast_to(af[:, None, :NF], (nb, FPH, NF)).reshape(r, NF)
        afi_b = jnp.broadcast_to(af[:, None, NF:], (nb, FPH, NF)).reshape(r, NF)
        vfr, vfi = vf[:, :NF], vf[:, NF:]
        orr = vfr * afr_b + vfi * afi_b
        oii = vfi * afr_b - vfr * afi_b
        oo = jnp.concatenate([orr, oii], axis=1)
        def dgt(x, y):
            return jax.lax.dot_general(
                x, y, (((1,), (1,)), ((), ())),
                preferred_element_type=jnp.float32)
        oh, om = _bsplit(oo)
        outw = (dgt(d2h_ref[...], om) + dgt(d2m_ref[...], oh)
                ) + dgt(d2h_ref[...], oh)
        for n in range(nb):
            out_ref[a + n] = outw[:, n * FPH:(n + 1) * FPH]

    if len(WAVES) == 1:
        a, b = WAVES[0]
        corr0, vf0 = stage_corr(a, b)
        s0 = stage_topk(corr0, a, b)
        stage_agg(s0, vf0, a, b)
    else:
        (a0, b0), (a1, b1) = WAVES
        corr0, vf0 = stage_corr(a0, b0)
        corr1, vf1 = stage_corr(a1, b1)
        s0 = stage_topk(corr0, a0, b0)
        stage_agg(s0, vf0, a0, b0)
        s1 = stage_topk(corr1, a1, b1)
        stage_agg(s1, vf1, a1, b1)


@jax.jit
def kernel(Q_in, K_in, V_in, t, W_v):
    B, N = Q_in.shape[0], Q_in.shape[1]
    NT = B * N
    Q = Q_in.reshape(NT, T, F)
    K = K_in.reshape(NT, T, F)
    V = V_in.reshape(NT, T, F)[:, :, :FPH]
    t3 = t.reshape(B, 1, T)
    ch, cm, dh, dm, d2h, d2m = (jnp.asarray(m) for m in _dft_mats())
    bpb = NPB // NB
    out, d5, w5 = pl.pallas_call(
        _kern,
        grid=(NT // NB,),
        compiler_params=pltpu.CompilerParams(
            dimension_semantics=("parallel",)),
        in_specs=[
            pl.BlockSpec((NB, T, F), lambda i: (i, 0, 0)),
            pl.BlockSpec((NB, T, F), lambda i: (i, 0, 0)),
            pl.BlockSpec((NB, T, FPH), lambda i: (i, 0, 0)),
            pl.BlockSpec((1, 1, T), lambda i: (i // bpb, 0, 0)),
            pl.BlockSpec((F + 1, FPH), lambda i: (0, 0)),
            pl.BlockSpec((T, NF2), lambda i: (0, 0)),
            pl.BlockSpec((T, NF2), lambda i: (0, 0)),
            pl.BlockSpec((NF2, T), lambda i: (0, 0)),
            pl.BlockSpec((NF2, T), lambda i: (0, 0)),
            pl.BlockSpec((T, NF2), lambda i: (0, 0)),
            pl.BlockSpec((T, NF2), lambda i: (0, 0)),
        ],
        out_specs=[
            pl.BlockSpec((NB, T, FPH), lambda i: (i, 0, 0)),
            pl.BlockSpec((NB * FPH, TOPK), lambda i: (i, 0)),
            pl.BlockSpec((NB * FPH, TOPK), lambda i: (i, 0)),
        ],
        out_shape=[
            jax.ShapeDtypeStruct((NT, T, FPH), jnp.float32),
            jax.ShapeDtypeStruct((NT * FPH, TOPK), jnp.int32),
            jax.ShapeDtypeStruct((NT * FPH, TOPK), jnp.float32),
        ],
    )(Q, K, V, t3, W_v, ch, cm, dh, dm, d2h, d2m)
    output = out.reshape(B, N, T, FPH)
    delay = d5.reshape(B, N, FPH, TOPK)
    tmp_corr = w5.reshape(B, N, FPH, TOPK)
    return output, delay, tmp_corr
